# baseline (speedup 1.0000x reference)
import dataclasses
import functools

import jax
import jax.numpy as jnp
from jax import lax
from jax.experimental import pallas as pl
from jax.experimental.pallas import tpu as pltpu
from jax.experimental.pallas import tpu_sc as plsc

N = 10000
NPAD = 10240
D = 256
EPS = 1e-16

E = 160000
EPT = E // 16
HALF = NPAD // 2
SB = 400
NS = EPT // SB
K = 16
DENW = 128
DQ = HALF // 128
DQP = 48
CPAD = 128
RPT = HALF // 16


def _pre_body(x_ref, ws_ref, wd_ref, as8_ref, ad8_ref, h_ref, al_ref):
    x = x_ref[...]
    h = jnp.dot(x, ws_ref[...], preferred_element_type=jnp.float32)
    h_ref[...] = h
    vp = jnp.dot(ws_ref[...], as8_ref[...], preferred_element_type=jnp.float32)
    vp = vp + jnp.dot(wd_ref[...], ad8_ref[...], preferred_element_type=jnp.float32)
    al_ref[...] = jnp.dot(x, vp, preferred_element_type=jnp.float32)


def _fin_pre_body(nlo_ref, nhi_ref, den_ref, b_ref, ws_ref, wd_ref,
                  as8_ref, ad8_ref, x1_ref, h_ref, al_ref):
    num = jnp.concatenate([nlo_ref[...], nhi_ref[...]], axis=1)
    den = den_ref[...]
    x1 = jnp.maximum(num / (den + EPS) + b_ref[...], 0.0)
    x1_ref[...] = x1
    h = jnp.dot(x1, ws_ref[...], preferred_element_type=jnp.float32)
    h_ref[...] = h
    vp = jnp.dot(ws_ref[...], as8_ref[...], preferred_element_type=jnp.float32)
    vp = vp + jnp.dot(wd_ref[...], ad8_ref[...], preferred_element_type=jnp.float32)
    al_ref[...] = jnp.dot(x1, vp, preferred_element_type=jnp.float32)


def _fin_body(nlo_ref, nhi_ref, den_ref, b_ref, x2_ref):
    num = jnp.concatenate([nlo_ref[...], nhi_ref[...]], axis=1)
    x2_ref[...] = num / (den_ref[...] + EPS) + b_ref[...]


_f32 = jnp.float32


def _pre(x, w_src, w_dst, as8, ad8):
    return pl.pallas_call(
        _pre_body,
        out_shape=(jax.ShapeDtypeStruct((NPAD, D), _f32),
                   jax.ShapeDtypeStruct((NPAD, 8), _f32)),
    )(x, w_src, w_dst, as8, ad8)


def _fin_pre(nlo, nhi, den, b, w_src, w_dst, as8, ad8):
    return pl.pallas_call(
        _fin_pre_body,
        out_shape=(jax.ShapeDtypeStruct((NPAD, D), _f32),
                   jax.ShapeDtypeStruct((NPAD, D), _f32),
                   jax.ShapeDtypeStruct((NPAD, 8), _f32)),
    )(nlo, nhi, den, b.reshape(1, D), w_src, w_dst, as8, ad8)


def _fin(nlo, nhi, den, b):
    return pl.pallas_call(
        _fin_body,
        out_shape=jax.ShapeDtypeStruct((NPAD, D), _f32),
    )(nlo, nhi, den, b.reshape(1, D))


def _den_col(den):
    return den.reshape(NPAD, 1)


def _edge_body(src_hbm, dst_hbm, als_hbm, ald_hbm, h_hbm,
               numl_hbm, numh_hbm, den_hbm,
               als_v, ald_v, src_v, dst_v, csrc_v, cdst_v, cex_v,
               rows0_v, rows1_v, stlo_v, sthi_v, den_v,
               idx0_v, idx1_v, idxd_v, idxq_v,
               accl_sh, acch_sh, dall_sh, sem0, sem1):
    cid = lax.axis_index("c")
    sid = lax.axis_index("s")
    lo = cid * HALF
    zero16 = jnp.zeros((16,), jnp.float32)
    zero16i = jnp.zeros((16,), jnp.int32)

    pltpu.sync_copy(als_hbm, als_v)
    pltpu.sync_copy(ald_hbm, ald_v)

    @pl.loop(0, SB + CPAD, step=16)
    def _(i):
        csrc_v[pl.ds(i, 16)] = zero16i
        cdst_v[pl.ds(i, 16)] = zero16i
        cex_v[pl.ds(i, 16)] = zero16

    @pl.loop(0, DQP)
    def _(r):
        @pl.loop(0, 128, step=16)
        def _(c):
            den_v[r, pl.ds(c, 16)] = zero16

    @pl.loop(0, K)
    def _(r):
        @pl.loop(0, DENW, step=16)
        def _(c):
            stlo_v[r, pl.ds(c, 16)] = zero16

    row0 = sid * RPT

    @pl.loop(0, RPT, step=K)
    def _(j):
        pltpu.sync_copy(stlo_v, accl_sh.at[pl.ds(row0 + j, K)])
        pltpu.sync_copy(stlo_v, acch_sh.at[pl.ds(row0 + j, K)])

    @pl.when(sid < 6)
    def _():
        pltpu.sync_copy(stlo_v.at[pl.ds(0, 8)], dall_sh.at[pl.ds(sid * 8, 8)])

    @pl.loop(0, DQP, step=16)
    def _(i):
        idxq_v[pl.ds(i, 16)] = jax.lax.iota(jnp.int32, 16) + i

    def _vmax(i, m):
        return jnp.maximum(m, als_v[pl.ds(i * 16, 16)])

    def _vmax2(i, m):
        return jnp.maximum(m, ald_v[pl.ds(i * 16, 16)])

    neg = jnp.full((16,), -3.0e38, jnp.float32)
    ms = jnp.max(lax.fori_loop(0, NPAD // 16, _vmax, neg), axis=0)
    md = jnp.max(lax.fori_loop(0, NPAD // 16, _vmax2, neg), axis=0)
    shift = jnp.maximum(ms + md, 0.0)

    plsc.subcore_barrier()

    base = sid * EPT

    def _issue(g, idx_v, rows_v, sem):
        @pl.loop(0, K, step=16)
        def _(i):
            idx_v[pl.ds(i, 16)] = csrc_v[pl.ds(g * K + i, 16)]
        pltpu.async_copy(h_hbm.at[idx_v], rows_v, sem)

    def _drain(rows_v, sem):
        pltpu.make_async_copy(h_hbm.at[pl.ds(0, K)], rows_v, sem).wait()

    def _process(g, rows_v):
        @pl.loop(0, K)
        def _(r):
            exv = plsc.load_gather(
                cex_v, [jnp.full((16,), g * K + r, jnp.int32)])

            @pl.loop(0, DENW, step=16)
            def _(c):
                stlo_v[r, pl.ds(c, 16)] = rows_v[r, pl.ds(c, 16)] * exv
                sthi_v[r, pl.ds(c, 16)] = (
                    rows_v[r, pl.ds(DENW + c, 16)] * exv)

        @pl.loop(0, K, step=16)
        def _(i):
            idxd_v[pl.ds(i, 16)] = cdst_v[pl.ds(g * K + i, 16)]
        pltpu.sync_copy(stlo_v, accl_sh.at[idxd_v], add=True)
        pltpu.sync_copy(sthi_v, acch_sh.at[idxd_v], add=True)

    @pl.loop(0, NS)
    def _stripe(s):
        off = base + s * SB
        pltpu.sync_copy(src_hbm.at[pl.ds(off, SB)], src_v)
        pltpu.sync_copy(dst_hbm.at[pl.ds(off, SB)], dst_v)

        def _compact(i, cnt):
            sv = src_v[pl.ds(i * 16, 16)]
            dv = dst_v[pl.ds(i * 16, 16)]
            a1 = plsc.load_gather(als_v, [sv])
            a2 = plsc.load_gather(ald_v, [dv])
            t = a1 + a2
            logit = jnp.where(t > 0, t, 0.2 * t)
            ex = jnp.exp(logit - shift)
            ldv = dv - lo
            mask = (ldv >= 0) & (ldv < HALF)
            plsc.addupdate_scatter(
                den_v, [lax.shift_right_logical(ldv, 7), ldv & 127], ex,
                mask=mask)
            plsc.store_compressed(csrc_v.at[pl.ds(cnt, 16)], sv, mask=mask)
            plsc.store_compressed(cdst_v.at[pl.ds(cnt, 16)], ldv, mask=mask)
            plsc.store_compressed(cex_v.at[pl.ds(cnt, 16)], ex, mask=mask)
            return cnt + jnp.sum(mask.astype(jnp.int32), axis=0)

        cnt = lax.fori_loop(0, SB // 16, _compact, jnp.int32(0))
        @pl.loop(0, 2 * K, step=16)
        def _(i):
            cex_v[pl.ds(cnt + i, 16)] = zero16

        _issue(0, idx0_v, rows0_v, sem0)
        _issue(1, idx1_v, rows1_v, sem1)

        def _pair(q, _):
            _drain(rows0_v, sem0)
            _process(2 * q, rows0_v)
            _issue(2 * q + 2, idx0_v, rows0_v, sem0)
            _drain(rows1_v, sem1)
            _process(2 * q + 1, rows1_v)
            _issue(2 * q + 3, idx1_v, rows1_v, sem1)
            return 0

        ng = (cnt + (K - 1)) // K
        lax.fori_loop(0, (ng + 1) // 2, _pair, 0)
        _drain(rows0_v, sem0)
        _drain(rows1_v, sem1)

    pltpu.sync_copy(den_v, dall_sh.at[idxq_v], add=True)
    plsc.subcore_barrier()

    @pl.loop(0, RPT, step=K)
    def _(j):
        pltpu.sync_copy(accl_sh.at[pl.ds(row0 + j, K)], stlo_v)
        pltpu.sync_copy(stlo_v, numl_hbm.at[pl.ds(lo + row0 + j, K)])
        pltpu.sync_copy(acch_sh.at[pl.ds(row0 + j, K)], sthi_v)
        pltpu.sync_copy(sthi_v, numh_hbm.at[pl.ds(lo + row0 + j, K)])

    r0 = sid * 8

    @pl.when(sid < 5)
    def _():
        pltpu.sync_copy(dall_sh.at[pl.ds(r0, 8)], stlo_v.at[pl.ds(0, 8)])
        pltpu.sync_copy(stlo_v.at[pl.ds(0, 8)],
                        den_hbm.at[pl.ds(cid * DQ + r0, 8)])


def _sc_compiler_params():
    cp = pltpu.CompilerParams()
    if "needs_layout_passes" in pltpu.CompilerParams.__dataclass_fields__:
        cp = dataclasses.replace(cp, needs_layout_passes=False)
    return cp


def _edge_sc(src, dst, als, ald, h):
    mesh = plsc.VectorSubcoreMesh(core_axis_name="c", subcore_axis_name="s")
    f = pl.kernel(
        _edge_body,
        out_type=(jax.ShapeDtypeStruct((NPAD, DENW), jnp.float32),
                  jax.ShapeDtypeStruct((NPAD, DENW), jnp.float32),
                  jax.ShapeDtypeStruct((2 * DQ, 128), jnp.float32)),
        mesh=mesh,
        compiler_params=_sc_compiler_params(),
        scratch_types=[
            pltpu.VMEM((NPAD,), jnp.float32),
            pltpu.VMEM((NPAD,), jnp.float32),
            pltpu.VMEM((SB,), jnp.int32),
            pltpu.VMEM((SB,), jnp.int32),
            pltpu.VMEM((SB + CPAD,), jnp.int32),
            pltpu.VMEM((SB + CPAD,), jnp.int32),
            pltpu.VMEM((SB + CPAD,), jnp.float32),
            pltpu.VMEM((K, D), jnp.float32),
            pltpu.VMEM((K, D), jnp.float32),
            pltpu.VMEM((K, DENW), jnp.float32),
            pltpu.VMEM((K, DENW), jnp.float32),
            pltpu.VMEM((DQP, 128), jnp.float32),
            pltpu.VMEM((K,), jnp.int32),
            pltpu.VMEM((K,), jnp.int32),
            pltpu.VMEM((K,), jnp.int32),
            pltpu.VMEM((DQP,), jnp.int32),
            pltpu.VMEM_SHARED((HALF, DENW), jnp.float32),
            pltpu.VMEM_SHARED((HALF, DENW), jnp.float32),
            pltpu.VMEM_SHARED((DQP, 128), jnp.float32),
            pltpu.SemaphoreType.DMA,
            pltpu.SemaphoreType.DMA,
        ],
    )
    return f(src, dst, als, ald, h)


def _pack8(a_src, a_dst):
    as8 = jnp.zeros((D, 8), _f32).at[:, 0].set(a_src)
    ad8 = jnp.zeros((D, 8), _f32).at[:, 1].set(a_dst)
    return as8, ad8


def kernel(x, edge_index, W1_src, W1_dst, a1_src, a1_dst, b1,
           W2_src, W2_dst, a2_src, a2_dst, b2):
    src = edge_index[0]
    dst = edge_index[1]
    xp = jnp.zeros((NPAD, D), _f32).at[:N].set(x)

    as8_1, ad8_1 = _pack8(a1_src, a1_dst)
    as8_2, ad8_2 = _pack8(a2_src, a2_dst)

    h1, al1 = _pre(xp, W1_src, W1_dst, as8_1, ad8_1)
    nlo1, nhi1, den1 = _edge_sc(src, dst, al1[:, 0], al1[:, 1], h1)
    x1, h2, al2 = _fin_pre(nlo1, nhi1, _den_col(den1), b1, W2_src, W2_dst,
                           as8_2, ad8_2)
    nlo2, nhi2, den2 = _edge_sc(src, dst, al2[:, 0], al2[:, 1], h2)
    x2 = _fin(nlo2, nhi2, _den_col(den2), b2)
    return jnp.concatenate([x1[:N], x2[:N]], axis=1)


# speedup vs baseline: 1.8246x; 1.8246x over previous
import dataclasses
import functools

import jax
import jax.numpy as jnp
from jax import lax
from jax.experimental import pallas as pl
from jax.experimental.pallas import tpu as pltpu
from jax.experimental.pallas import tpu_sc as plsc

N = 10000
NPAD = 10240
D = 256
EPS = 1e-16

E = 160000
EPT = E // 16
HALF = NPAD // 2
SB = 1680
NS = -(-EPT // SB)
PADE = NS * SB - EPT
K = 16
DENW = 128
DQ = HALF // 128
DQP = 48
CPAD = 64
RPT = HALF // 16


def _pre_body(x_ref, ws_ref, wd_ref, as8_ref, ad8_ref, h_ref, al_ref):
    x = x_ref[...]
    h = jnp.dot(x, ws_ref[...], preferred_element_type=jnp.float32)
    h_ref[...] = h
    vp = jnp.dot(ws_ref[...], as8_ref[...], preferred_element_type=jnp.float32)
    vp = vp + jnp.dot(wd_ref[...], ad8_ref[...], preferred_element_type=jnp.float32)
    al_ref[...] = jnp.dot(x, vp, preferred_element_type=jnp.float32)


def _fin_pre_body(nlo_ref, nhi_ref, den_ref, b_ref, ws_ref, wd_ref,
                  as8_ref, ad8_ref, x1_ref, h_ref, al_ref):
    num = jnp.concatenate([nlo_ref[...], nhi_ref[...]], axis=1)
    den = den_ref[...]
    x1 = jnp.maximum(num / (den + EPS) + b_ref[...], 0.0)
    x1_ref[...] = x1
    h = jnp.dot(x1, ws_ref[...], preferred_element_type=jnp.float32)
    h_ref[...] = h
    vp = jnp.dot(ws_ref[...], as8_ref[...], preferred_element_type=jnp.float32)
    vp = vp + jnp.dot(wd_ref[...], ad8_ref[...], preferred_element_type=jnp.float32)
    al_ref[...] = jnp.dot(x1, vp, preferred_element_type=jnp.float32)


def _fin_body(nlo_ref, nhi_ref, den_ref, b_ref, x2_ref):
    num = jnp.concatenate([nlo_ref[...], nhi_ref[...]], axis=1)
    x2_ref[...] = num / (den_ref[...] + EPS) + b_ref[...]


_f32 = jnp.float32


def _pre(x, w_src, w_dst, as8, ad8):
    return pl.pallas_call(
        _pre_body,
        out_shape=(jax.ShapeDtypeStruct((NPAD, D), _f32),
                   jax.ShapeDtypeStruct((NPAD, 8), _f32)),
    )(x, w_src, w_dst, as8, ad8)


def _fin_pre(nlo, nhi, den, b, w_src, w_dst, as8, ad8):
    return pl.pallas_call(
        _fin_pre_body,
        out_shape=(jax.ShapeDtypeStruct((NPAD, D), _f32),
                   jax.ShapeDtypeStruct((NPAD, D), _f32),
                   jax.ShapeDtypeStruct((NPAD, 8), _f32)),
    )(nlo, nhi, den, b.reshape(1, D), w_src, w_dst, as8, ad8)


def _fin(nlo, nhi, den, b):
    return pl.pallas_call(
        _fin_body,
        out_shape=jax.ShapeDtypeStruct((NPAD, D), _f32),
    )(nlo, nhi, den, b.reshape(1, D))


def _den_col(den):
    return den.reshape(NPAD, 1)


def _edge_body(src_hbm, dst_hbm, als_hbm, ald_hbm, h_hbm,
               numl_hbm, numh_hbm, den_hbm,
               als_v, ald_v, src_v, dst_v, csrc_v, cdst_v, cex_v,
               rows0_v, rows1_v, stlo_v, sthi_v, den_v,
               idx0_v, idx1_v, idxd_v, idxq_v,
               accl_sh, acch_sh, dall_sh, sem0, sem1):
    cid = lax.axis_index("c")
    sid = lax.axis_index("s")
    lo = cid * HALF
    zero16 = jnp.zeros((16,), jnp.float32)
    zero16i = jnp.zeros((16,), jnp.int32)

    pltpu.sync_copy(als_hbm, als_v)
    pltpu.sync_copy(ald_hbm, ald_v)

    @pl.loop(0, SB + CPAD, step=16)
    def _(i):
        csrc_v[pl.ds(i, 16)] = zero16i
        cdst_v[pl.ds(i, 16)] = zero16i
        cex_v[pl.ds(i, 16)] = zero16

    @pl.loop(0, DQP)
    def _(r):
        @pl.loop(0, 128, step=16)
        def _(c):
            den_v[r, pl.ds(c, 16)] = zero16

    @pl.loop(0, K)
    def _(r):
        @pl.loop(0, DENW, step=16)
        def _(c):
            stlo_v[r, pl.ds(c, 16)] = zero16

    row0 = sid * RPT

    @pl.loop(0, RPT, step=K)
    def _(j):
        pltpu.sync_copy(stlo_v, accl_sh.at[pl.ds(row0 + j, K)])
        pltpu.sync_copy(stlo_v, acch_sh.at[pl.ds(row0 + j, K)])

    @pl.when(sid < 6)
    def _():
        pltpu.sync_copy(stlo_v.at[pl.ds(0, 8)], dall_sh.at[pl.ds(sid * 8, 8)])

    @pl.loop(0, DQP, step=16)
    def _(i):
        idxq_v[pl.ds(i, 16)] = jax.lax.iota(jnp.int32, 16) + i

    def _vmax(i, m):
        return jnp.maximum(m, als_v[pl.ds(i * 16, 16)])

    def _vmax2(i, m):
        return jnp.maximum(m, ald_v[pl.ds(i * 16, 16)])

    neg = jnp.full((16,), -3.0e38, jnp.float32)
    ms = jnp.max(lax.fori_loop(0, NPAD // 16, _vmax, neg), axis=0)
    md = jnp.max(lax.fori_loop(0, NPAD // 16, _vmax2, neg), axis=0)
    shift = jnp.maximum(ms + md, 0.0)

    plsc.subcore_barrier()

    base = sid * EPT

    def _issue(g, idx_v, rows_v, sem):
        @pl.loop(0, K, step=16)
        def _(i):
            idx_v[pl.ds(i, 16)] = csrc_v[pl.ds(g * K + i, 16)]
        pltpu.async_copy(h_hbm.at[idx_v], rows_v, sem)

    def _drain(rows_v, sem):
        pltpu.make_async_copy(h_hbm.at[pl.ds(0, K)], rows_v, sem).wait()

    def _process(g, rows_v):
        @pl.loop(0, K)
        def _(r):
            exv = plsc.load_gather(
                cex_v, [jnp.full((16,), g * K + r, jnp.int32)])

            @pl.loop(0, DENW, step=16)
            def _(c):
                stlo_v[r, pl.ds(c, 16)] = rows_v[r, pl.ds(c, 16)] * exv
                sthi_v[r, pl.ds(c, 16)] = (
                    rows_v[r, pl.ds(DENW + c, 16)] * exv)

        @pl.loop(0, K, step=16)
        def _(i):
            idxd_v[pl.ds(i, 16)] = cdst_v[pl.ds(g * K + i, 16)]
        pltpu.sync_copy(stlo_v, accl_sh.at[idxd_v], add=True)
        pltpu.sync_copy(sthi_v, acch_sh.at[idxd_v], add=True)

    @pl.loop(0, NS)
    def _stripe(s):
        off = base + s * SB
        pltpu.sync_copy(src_hbm.at[pl.ds(off, SB)], src_v)
        pltpu.sync_copy(dst_hbm.at[pl.ds(off, SB)], dst_v)

        def _compact(i, cnt):
            sv = src_v[pl.ds(i * 16, 16)]
            dv = dst_v[pl.ds(i * 16, 16)]
            a1 = plsc.load_gather(als_v, [sv])
            a2 = plsc.load_gather(ald_v, [dv])
            t = a1 + a2
            logit = jnp.where(t > 0, t, 0.2 * t)
            ex = jnp.exp(logit - shift)
            ldv = dv - lo
            iv = (jnp.full((16,), s * SB + i * 16, jnp.int32)
                  + jax.lax.iota(jnp.int32, 16))
            mask = (ldv >= 0) & (ldv < HALF) & (iv < EPT)
            plsc.addupdate_scatter(
                den_v, [lax.shift_right_logical(ldv, 7), ldv & 127], ex,
                mask=mask)
            plsc.store_compressed(csrc_v.at[pl.ds(cnt, 16)], sv, mask=mask)
            plsc.store_compressed(cdst_v.at[pl.ds(cnt, 16)], ldv, mask=mask)
            plsc.store_compressed(cex_v.at[pl.ds(cnt, 16)], ex, mask=mask)
            return cnt + jnp.sum(mask.astype(jnp.int32), axis=0)

        cnt = lax.fori_loop(0, SB // 16, _compact, jnp.int32(0))
        @pl.loop(0, 2 * K, step=16)
        def _(i):
            cex_v[pl.ds(cnt + i, 16)] = zero16

        _issue(0, idx0_v, rows0_v, sem0)
        _issue(1, idx1_v, rows1_v, sem1)

        def _pair(q, _):
            _drain(rows0_v, sem0)
            _process(2 * q, rows0_v)
            _issue(2 * q + 2, idx0_v, rows0_v, sem0)
            _drain(rows1_v, sem1)
            _process(2 * q + 1, rows1_v)
            _issue(2 * q + 3, idx1_v, rows1_v, sem1)
            return 0

        ng = (cnt + (K - 1)) // K
        lax.fori_loop(0, (ng + 1) // 2, _pair, 0)
        _drain(rows0_v, sem0)
        _drain(rows1_v, sem1)

    pltpu.sync_copy(den_v, dall_sh.at[idxq_v], add=True)
    plsc.subcore_barrier()

    @pl.loop(0, RPT, step=K)
    def _(j):
        pltpu.sync_copy(accl_sh.at[pl.ds(row0 + j, K)], stlo_v)
        pltpu.sync_copy(stlo_v, numl_hbm.at[pl.ds(lo + row0 + j, K)])
        pltpu.sync_copy(acch_sh.at[pl.ds(row0 + j, K)], sthi_v)
        pltpu.sync_copy(sthi_v, numh_hbm.at[pl.ds(lo + row0 + j, K)])

    r0 = sid * 8

    @pl.when(sid < 5)
    def _():
        pltpu.sync_copy(dall_sh.at[pl.ds(r0, 8)], stlo_v.at[pl.ds(0, 8)])
        pltpu.sync_copy(stlo_v.at[pl.ds(0, 8)],
                        den_hbm.at[pl.ds(cid * DQ + r0, 8)])


def _sc_compiler_params():
    cp = pltpu.CompilerParams()
    if "needs_layout_passes" in pltpu.CompilerParams.__dataclass_fields__:
        cp = dataclasses.replace(cp, needs_layout_passes=False)
    return cp


def _edge_sc(src, dst, als, ald, h):
    mesh = plsc.VectorSubcoreMesh(core_axis_name="c", subcore_axis_name="s")
    f = pl.kernel(
        _edge_body,
        out_type=(jax.ShapeDtypeStruct((NPAD, DENW), jnp.float32),
                  jax.ShapeDtypeStruct((NPAD, DENW), jnp.float32),
                  jax.ShapeDtypeStruct((2 * DQ, 128), jnp.float32)),
        mesh=mesh,
        compiler_params=_sc_compiler_params(),
        scratch_types=[
            pltpu.VMEM((NPAD,), jnp.float32),
            pltpu.VMEM((NPAD,), jnp.float32),
            pltpu.VMEM((SB,), jnp.int32),
            pltpu.VMEM((SB,), jnp.int32),
            pltpu.VMEM((SB + CPAD,), jnp.int32),
            pltpu.VMEM((SB + CPAD,), jnp.int32),
            pltpu.VMEM((SB + CPAD,), jnp.float32),
            pltpu.VMEM((K, D), jnp.float32),
            pltpu.VMEM((K, D), jnp.float32),
            pltpu.VMEM((K, DENW), jnp.float32),
            pltpu.VMEM((K, DENW), jnp.float32),
            pltpu.VMEM((DQP, 128), jnp.float32),
            pltpu.VMEM((K,), jnp.int32),
            pltpu.VMEM((K,), jnp.int32),
            pltpu.VMEM((K,), jnp.int32),
            pltpu.VMEM((DQP,), jnp.int32),
            pltpu.VMEM_SHARED((HALF, DENW), jnp.float32),
            pltpu.VMEM_SHARED((HALF, DENW), jnp.float32),
            pltpu.VMEM_SHARED((DQP, 128), jnp.float32),
            pltpu.SemaphoreType.DMA,
            pltpu.SemaphoreType.DMA,
        ],
    )
    return f(src, dst, als, ald, h)


def _pack8(a_src, a_dst):
    as8 = jnp.zeros((D, 8), _f32).at[:, 0].set(a_src)
    ad8 = jnp.zeros((D, 8), _f32).at[:, 1].set(a_dst)
    return as8, ad8


def kernel(x, edge_index, W1_src, W1_dst, a1_src, a1_dst, b1,
           W2_src, W2_dst, a2_src, a2_dst, b2):
    pad = jnp.zeros((PADE,), edge_index.dtype)
    src = jnp.concatenate([edge_index[0], pad])
    dst = jnp.concatenate([edge_index[1], pad])
    xp = jnp.zeros((NPAD, D), _f32).at[:N].set(x)

    as8_1, ad8_1 = _pack8(a1_src, a1_dst)
    as8_2, ad8_2 = _pack8(a2_src, a2_dst)

    h1, al1 = _pre(xp, W1_src, W1_dst, as8_1, ad8_1)
    nlo1, nhi1, den1 = _edge_sc(src, dst, al1[:, 0], al1[:, 1], h1)
    x1, h2, al2 = _fin_pre(nlo1, nhi1, _den_col(den1), b1, W2_src, W2_dst,
                           as8_2, ad8_2)
    nlo2, nhi2, den2 = _edge_sc(src, dst, al2[:, 0], al2[:, 1], h2)
    x2 = _fin(nlo2, nhi2, _den_col(den2), b2)
    return jnp.concatenate([x1[:N], x2[:N]], axis=1)
